# baseline (speedup 1.0000x reference)
import functools

import jax
import jax.numpy as jnp
from jax import lax
from jax.experimental import pallas as pl
from jax.experimental.pallas import tpu as pltpu
from jax.experimental.pallas import tpu_sc as plsc

K = 64


def _sc_select(z, kk):
    m, n = z.shape
    info = plsc.get_sparse_core_info()
    NC, NS, L = info.num_cores, info.num_subcores, info.num_lanes
    NW = NC * NS
    rpw = m // NW
    csz = 128
    nchunks = n // csz
    BUF = 2048

    mesh = plsc.VectorSubcoreMesh(core_axis_name="c", subcore_axis_name="s")

    def _count_ge(load, nv, mid):
        def b(i, c):
            return c + plsc.all_reduce_population_count(load(i) >= mid)
        return lax.fori_loop(0, nv, b, jnp.zeros((L,), jnp.int32))

    def _bisect(load, nv, kk_, lo, hi, iters):
        def b(_, lh):
            lo, hi = lh
            mid = lo + lax.shift_right_logical(hi - lo, 1)
            ge = _count_ge(load, nv, mid) >= kk_
            return jnp.where(ge, mid, lo), jnp.where(ge, hi, mid)
        return lax.fori_loop(0, iters, b, (lo, hi))[0]

    NACC = 8
    VPB = n // L // NACC
    CBUF = 256
    NCV = CBUF // L

    @functools.partial(
        pl.kernel, mesh=mesh,
        compiler_params=pltpu.CompilerParams(needs_layout_passes=False),
        out_type=jax.ShapeDtypeStruct((m,), jnp.int32),
        scratch_types=[
            pltpu.VMEM((n // csz, csz), jnp.float32),
            pltpu.VMEM((n // csz, csz), jnp.float32),
            pltpu.VMEM((NCV, L), jnp.int32),
            pltpu.VMEM((rpw,), jnp.int32),
            pltpu.SemaphoreType.DMA,
            pltpu.SemaphoreType.DMA,
        ],
    )
    def sel(z_hbm, t_hbm, zrow0, zrow1, cand, tout, sem0, sem1):
        wid = lax.axis_index("s") * NC + lax.axis_index("c")
        base = wid * rpw
        iota = lax.iota(jnp.int32, L)
        zeros = jnp.zeros((L,), jnp.int32)
        infty = jnp.full((L,), 0x7F800000, jnp.int32)
        kkv = jnp.full((L,), kk, jnp.int32)
        vpr = csz // L

        def vload(zrow, i):
            return plsc.bitcast(zrow[i // vpr, pl.ds((i % vpr) * L, L)],
                                jnp.int32)

        def process(zrow):
            def cm(i, accs):
                vs = [vload(zrow, i * NACC + u) for u in range(NACC)]
                return tuple(jnp.maximum(a, v) for a, v in zip(accs, vs))
            accs = lax.fori_loop(0, VPB, cm, (zeros,) * NACC)

            def bis_acc(_, lh):
                lo, hi = lh
                mid = lo + lax.shift_right_logical(hi - lo, 1)
                cnt = zeros
                for a in accs:
                    cnt = cnt + plsc.all_reduce_population_count(a >= mid)
                ge = cnt >= kkv + 1
                return jnp.where(ge, mid, lo), jnp.where(ge, hi, mid)
            m65 = lax.fori_loop(0, 16, bis_acc, (zeros, infty))[0]
            c0 = jnp.maximum(m65, 1)

            for i in range(NCV):
                cand[i, :] = zeros

            def comp(i, off):
                vs = [vload(zrow, i * NACC + u) for u in range(NACC)]
                msks = [v >= c0 for v in vs]
                pcs = [plsc.all_reduce_population_count(k) for k in msks]
                offs = [off]
                for u in range(NACC - 1):
                    offs.append(offs[-1] + pcs[u])
                for u in range(NACC):
                    ranks = plsc.cumsum(jnp.where(msks[u], 1, 0))
                    idx = offs[u] + ranks - 1
                    ok = jnp.logical_and(msks[u], idx < CBUF)
                    plsc.store_scatter(cand, [idx // L, idx % L], vs[u],
                                       mask=ok)
                return offs[-1] + pcs[-1]
            nc = lax.fori_loop(0, VPB, comp, zeros)

            cregs = tuple(cand[i, :] for i in range(NCV))

            def bis_cand(_, lh):
                lo, hi = lh
                mid = lo + lax.shift_right_logical(hi - lo, 1)
                cnt = zeros
                for cv in cregs:
                    cnt = cnt + plsc.all_reduce_population_count(cv >= mid)
                ge = cnt >= kkv
                return jnp.where(ge, mid, lo), jnp.where(ge, hi, mid)
            t = lax.fori_loop(0, 31, bis_cand, (c0, infty))[0]
            return jnp.where(nc >= kkv, t, zeros)

        def start(r, zrow, sem):
            pltpu.async_copy(z_hbm.at[base + r], zrow, sem)

        def wait(zrow, sem):
            pltpu.make_async_copy(z_hbm.at[base], zrow, sem).wait()

        start(0, zrow0, sem0)

        def pair_body(b, tacc):
            r = 2 * b
            wait(zrow0, sem0)
            start(r + 1, zrow1, sem1)
            t0 = process(zrow0)
            tacc = jnp.where(iota == (r % L), t0, tacc)
            wait(zrow1, sem1)

            @pl.when(b < rpw // 2 - 1)
            def _prefetch():
                start(r + 2, zrow0, sem0)
            t1 = process(zrow1)
            tacc = jnp.where(iota == ((r + 1) % L), t1, tacc)

            @pl.when((r + 1) % L == L - 1)
            def _flush():
                tout[pl.ds((r // L) * L, L)] = tacc
            return tacc

        lax.fori_loop(0, rpw // 2, pair_body, zeros)
        pltpu.sync_copy(tout, t_hbm.at[pl.ds(base, rpw)])

    return sel(z.reshape(m, n // csz, csz))

_NT = (((1,), (1,)), ((), ()))


def _encode_body(x_ref, w_ref, b_ref, o_ref):
    acc = jax.lax.dot_general(x_ref[...], w_ref[...], _NT,
                              preferred_element_type=jnp.float32)
    o_ref[...] = jnp.maximum(acc + b_ref[...], 0.0)


def _encode(x, w_enc, b_enc, bm, bn, sm=None, roff=0):
    m, k = x.shape
    sm = m if sm is None else sm
    n = w_enc.shape[0]
    grid = (n // bn, sm // bm)
    return pl.pallas_call(
        _encode_body,
        grid=grid,
        in_specs=[
            pl.BlockSpec((bm, k), lambda j, i: (i + roff, 0)),
            pl.BlockSpec((bn, k), lambda j, i: (j, 0)),
            pl.BlockSpec((1, bn), lambda j, i: (0, j)),
        ],
        out_specs=pl.BlockSpec((bm, bn), lambda j, i: (i, j)),
        out_shape=jax.ShapeDtypeStruct((sm, n), jnp.float32),
    )(x, w_enc, b_enc)


def _select_body(z_ref, t_ref, *, kk):
    z = z_ref[...].view(jnp.int32)
    rows = z.shape[0]
    lo = jnp.zeros((rows, 1), jnp.int32)
    hi = jnp.full((rows, 1), 0x7F800000, jnp.int32)

    def body(_, carry):
        lo, hi = carry
        mid = lo + (hi - lo) // 2
        cnt = jnp.sum((z >= mid).astype(jnp.int32), axis=1, keepdims=True)
        ge = cnt >= kk
        return jnp.where(ge, mid, lo), jnp.where(ge, hi, mid)

    lo, hi = jax.lax.fori_loop(0, 31, body, (lo, hi))
    t_ref[...] = lo


def _select(z, bm, kk):
    m, n = z.shape
    return pl.pallas_call(
        functools.partial(_select_body, kk=kk),
        grid=(m // bm,),
        in_specs=[pl.BlockSpec((bm, n), lambda i: (i, 0))],
        out_specs=pl.BlockSpec((bm, 1), lambda i: (i, 0)),
        out_shape=jax.ShapeDtypeStruct((m, 1), jnp.int32),
    )(z)


def _decode_body(z_ref, t_ref, w_ref, b_ref, o_ref):
    kidx = pl.program_id(1)
    zi = z_ref[...].view(jnp.int32)
    zm = jnp.where(zi >= t_ref[...], z_ref[...], 0.0)
    part = jax.lax.dot_general(zm, w_ref[...], _NT,
                               preferred_element_type=jnp.float32)

    @pl.when(kidx == 0)
    def _init():
        o_ref[...] = part + b_ref[...]

    @pl.when(kidx > 0)
    def _acc():
        o_ref[...] += part


def _decode(z, t, w_dec, b_dec, bm, bk):
    m, n = z.shape
    d = w_dec.shape[0]
    grid = (m // bm, n // bk)
    return pl.pallas_call(
        _decode_body,
        grid=grid,
        in_specs=[
            pl.BlockSpec((bm, bk), lambda i, j: (i, j)),
            pl.BlockSpec((bm, 1), lambda i, j: (i, 0)),
            pl.BlockSpec((d, bk), lambda i, j: (0, j)),
            pl.BlockSpec((1, d), lambda i, j: (0, 0)),
        ],
        out_specs=pl.BlockSpec((bm, d), lambda i, j: (i, 0)),
        out_shape=jax.ShapeDtypeStruct((m, d), jnp.float32),
    )(z, t, w_dec, b_dec)


def _decode_slab_body(z_ref, t_ref, w_ref, b_ref, obuf_ref, o_ref):
    _decode_body(z_ref, t_ref, w_ref, b_ref, o_ref)


def _decode_slab(z, t, w_dec, b_dec, out_buf, roff, bm, bk):
    sm, n = z.shape
    mfull, d = out_buf.shape
    grid = (sm // bm, n // bk)
    return pl.pallas_call(
        _decode_slab_body,
        grid=grid,
        in_specs=[
            pl.BlockSpec((bm, bk), lambda i, j: (i, j)),
            pl.BlockSpec((bm, 1), lambda i, j: (i, 0)),
            pl.BlockSpec((d, bk), lambda i, j: (0, j)),
            pl.BlockSpec((1, d), lambda i, j: (0, 0)),
            pl.BlockSpec(memory_space=pltpu.MemorySpace.HBM),
        ],
        out_specs=pl.BlockSpec((bm, d), lambda i, j: (i + roff, 0)),
        out_shape=jax.ShapeDtypeStruct((mfull, d), jnp.float32),
        input_output_aliases={4: 0},
    )(z, t, w_dec, b_dec, out_buf)


def kernel(x, W_enc, b_enc, W_dec, b_dec):
    m, act = x.shape
    dict_size = W_enc.shape[0]
    b_enc2 = b_enc.reshape(1, dict_size)
    b_dec2 = b_dec.reshape(1, act)

    bm_e = min(512, m)
    bn_e = min(1024, dict_size)
    bm_d = min(1024, m)
    bk_d = min(512, dict_size)

    if m % 8192 == 0 and dict_size % 2048 == 0:
        nslab = 4
        sm = m // nslab
        bme, bmd = min(bm_e, sm), min(bm_d, sm)
        out = jnp.zeros((m, act), jnp.float32)
        for s in range(nslab):
            zs = _encode(x, W_enc, b_enc2, bme, bn_e,
                         sm=sm, roff=s * (sm // bme))
            ts = _sc_select(zs, K).reshape(sm, 1)
            out = _decode_slab(zs, ts, W_dec, b_dec2, out,
                               s * (sm // bmd), bmd, bk_d)
        return out

    z = _encode(x, W_enc, b_enc2, bm_e, bn_e)
    t = _select(z, min(128, m), K)
    return _decode(z, t, W_dec, b_dec2, bm_d, bk_d)


# speedup vs baseline: 1.1991x; 1.1991x over previous
import functools

import jax
import jax.numpy as jnp
from jax import lax
from jax.experimental import pallas as pl
from jax.experimental.pallas import tpu as pltpu
from jax.experimental.pallas import tpu_sc as plsc

K = 64


def _sc_select(z, kk):
    m, n = z.shape
    info = plsc.get_sparse_core_info()
    NC, NS, L = info.num_cores, info.num_subcores, info.num_lanes
    NW = NC * NS
    rpw = m // NW
    csz = 128
    nchunks = n // csz
    BUF = 2048

    mesh = plsc.VectorSubcoreMesh(core_axis_name="c", subcore_axis_name="s")

    def _count_ge(load, nv, mid):
        def b(i, c):
            return c + plsc.all_reduce_population_count(load(i) >= mid)
        return lax.fori_loop(0, nv, b, jnp.zeros((L,), jnp.int32))

    def _bisect(load, nv, kk_, lo, hi, iters):
        def b(_, lh):
            lo, hi = lh
            mid = lo + lax.shift_right_logical(hi - lo, 1)
            ge = _count_ge(load, nv, mid) >= kk_
            return jnp.where(ge, mid, lo), jnp.where(ge, hi, mid)
        return lax.fori_loop(0, iters, b, (lo, hi))[0]

    NACC = 8
    VPB = n // L // NACC
    CBUF = 256
    NCV = CBUF // L

    @functools.partial(
        pl.kernel, mesh=mesh,
        compiler_params=pltpu.CompilerParams(needs_layout_passes=False),
        out_type=jax.ShapeDtypeStruct((m,), jnp.int32),
        scratch_types=[
            pltpu.VMEM((n // csz, csz), jnp.float32),
            pltpu.VMEM((n // csz, csz), jnp.float32),
            pltpu.VMEM((NCV, L), jnp.int32),
            pltpu.VMEM((rpw,), jnp.int32),
            pltpu.SemaphoreType.DMA,
            pltpu.SemaphoreType.DMA,
        ],
    )
    def sel(z_hbm, t_hbm, zrow0, zrow1, cand, tout, sem0, sem1):
        wid = lax.axis_index("s") * NC + lax.axis_index("c")
        base = wid * rpw
        iota = lax.iota(jnp.int32, L)
        zeros = jnp.zeros((L,), jnp.int32)
        infty = jnp.full((L,), 0x7F800000, jnp.int32)
        kkv = jnp.full((L,), kk, jnp.int32)
        vpr = csz // L

        def vload(zrow, i):
            return plsc.bitcast(zrow[i // vpr, pl.ds((i % vpr) * L, L)],
                                jnp.int32)

        def process(zrow):
            def cm(i, accs):
                vs = [vload(zrow, i * NACC + u) for u in range(NACC)]
                return tuple(jnp.maximum(a, v) for a, v in zip(accs, vs))
            accs = lax.fori_loop(0, VPB, cm, (zeros,) * NACC)

            def bis_acc(_, lh):
                lo, hi = lh
                mid = lo + lax.shift_right_logical(hi - lo, 1)
                cnt = zeros
                for a in accs:
                    cnt = cnt + plsc.all_reduce_population_count(a >= mid)
                ge = cnt >= kkv + 1
                return jnp.where(ge, mid, lo), jnp.where(ge, hi, mid)
            m65 = lax.fori_loop(0, 16, bis_acc, (zeros, infty))[0]
            c0 = jnp.maximum(m65, 1)

            for i in range(NCV):
                cand[i, :] = zeros

            def comp(i, off):
                vs = [vload(zrow, i * NACC + u) for u in range(NACC)]
                msks = [v >= c0 for v in vs]
                pcs = [plsc.all_reduce_population_count(k) for k in msks]
                offs = [off]
                for u in range(NACC - 1):
                    offs.append(offs[-1] + pcs[u])
                for u in range(NACC):
                    ranks = plsc.cumsum(jnp.where(msks[u], 1, 0))
                    idx = offs[u] + ranks - 1
                    ok = jnp.logical_and(msks[u], idx < CBUF)
                    plsc.store_scatter(cand, [idx // L, idx % L], vs[u],
                                       mask=ok)
                return offs[-1] + pcs[-1]
            nc = lax.fori_loop(0, VPB, comp, zeros)

            cregs = tuple(cand[i, :] for i in range(NCV))

            def bis_cand(_, lh):
                lo, hi = lh
                mid = lo + lax.shift_right_logical(hi - lo, 1)
                cnt = zeros
                for cv in cregs:
                    cnt = cnt + plsc.all_reduce_population_count(cv >= mid)
                ge = cnt >= kkv
                return jnp.where(ge, mid, lo), jnp.where(ge, hi, mid)
            t = lax.fori_loop(0, 31, bis_cand, (c0, infty))[0]
            return jnp.where(nc >= kkv, t, zeros)

        def start(r, zrow, sem):
            pltpu.async_copy(z_hbm.at[base + r], zrow, sem)

        def wait(zrow, sem):
            pltpu.make_async_copy(z_hbm.at[base], zrow, sem).wait()

        start(0, zrow0, sem0)

        def pair_body(b, tacc):
            r = 2 * b
            wait(zrow0, sem0)
            start(r + 1, zrow1, sem1)
            t0 = process(zrow0)
            tacc = jnp.where(iota == (r % L), t0, tacc)
            wait(zrow1, sem1)

            @pl.when(b < rpw // 2 - 1)
            def _prefetch():
                start(r + 2, zrow0, sem0)
            t1 = process(zrow1)
            tacc = jnp.where(iota == ((r + 1) % L), t1, tacc)

            @pl.when((r + 1) % L == L - 1)
            def _flush():
                tout[pl.ds((r // L) * L, L)] = tacc
            return tacc

        lax.fori_loop(0, rpw // 2, pair_body, zeros)
        pltpu.sync_copy(tout, t_hbm.at[pl.ds(base, rpw)])

    return sel(z.reshape(m, n // csz, csz))

_NT = (((1,), (1,)), ((), ()))


def _encode_body(x_ref, w_ref, b_ref, o_ref):
    acc = jax.lax.dot_general(x_ref[...], w_ref[...], _NT,
                              preferred_element_type=jnp.float32)
    o_ref[...] = jnp.maximum(acc + b_ref[...], 0.0)


def _encode(x, w_enc, b_enc, bm, bn, sm=None, roff=0):
    m, k = x.shape
    sm = m if sm is None else sm
    n = w_enc.shape[0]
    grid = (n // bn, sm // bm)
    return pl.pallas_call(
        _encode_body,
        grid=grid,
        in_specs=[
            pl.BlockSpec((bm, k), lambda j, i: (i + roff, 0)),
            pl.BlockSpec((bn, k), lambda j, i: (j, 0)),
            pl.BlockSpec((1, bn), lambda j, i: (0, j)),
        ],
        out_specs=pl.BlockSpec((bm, bn), lambda j, i: (i, j)),
        out_shape=jax.ShapeDtypeStruct((sm, n), jnp.float32),
    )(x, w_enc, b_enc)


def _select_body(z_ref, t_ref, *, kk):
    z = z_ref[...].view(jnp.int32)
    rows = z.shape[0]
    lo = jnp.zeros((rows, 1), jnp.int32)
    hi = jnp.full((rows, 1), 0x7F800000, jnp.int32)

    def body(_, carry):
        lo, hi = carry
        mid = lo + (hi - lo) // 2
        cnt = jnp.sum((z >= mid).astype(jnp.int32), axis=1, keepdims=True)
        ge = cnt >= kk
        return jnp.where(ge, mid, lo), jnp.where(ge, hi, mid)

    lo, hi = jax.lax.fori_loop(0, 31, body, (lo, hi))
    t_ref[...] = lo


def _select(z, bm, kk):
    m, n = z.shape
    return pl.pallas_call(
        functools.partial(_select_body, kk=kk),
        grid=(m // bm,),
        in_specs=[pl.BlockSpec((bm, n), lambda i: (i, 0))],
        out_specs=pl.BlockSpec((bm, 1), lambda i: (i, 0)),
        out_shape=jax.ShapeDtypeStruct((m, 1), jnp.int32),
    )(z)


def _decode_body(z_ref, t_ref, w_ref, b_ref, o_ref):
    kidx = pl.program_id(1)
    zi = z_ref[...].view(jnp.int32)
    zm = jnp.where(zi >= t_ref[...], z_ref[...], 0.0)
    part = jax.lax.dot_general(zm, w_ref[...], _NT,
                               preferred_element_type=jnp.float32)

    @pl.when(kidx == 0)
    def _init():
        o_ref[...] = part + b_ref[...]

    @pl.when(kidx > 0)
    def _acc():
        o_ref[...] += part


def _decode(z, t, w_dec, b_dec, bm, bk):
    m, n = z.shape
    d = w_dec.shape[0]
    grid = (m // bm, n // bk)
    return pl.pallas_call(
        _decode_body,
        grid=grid,
        in_specs=[
            pl.BlockSpec((bm, bk), lambda i, j: (i, j)),
            pl.BlockSpec((bm, 1), lambda i, j: (i, 0)),
            pl.BlockSpec((d, bk), lambda i, j: (0, j)),
            pl.BlockSpec((1, d), lambda i, j: (0, 0)),
        ],
        out_specs=pl.BlockSpec((bm, d), lambda i, j: (i, 0)),
        out_shape=jax.ShapeDtypeStruct((m, d), jnp.float32),
    )(z, t, w_dec, b_dec)


def _decode_slab_body(z_ref, t_ref, w_ref, b_ref, obuf_ref, o_ref):
    _decode_body(z_ref, t_ref, w_ref, b_ref, o_ref)


def _decode_slab(z, t, w_dec, b_dec, out_buf, roff, bm, bk):
    sm, n = z.shape
    mfull, d = out_buf.shape
    grid = (sm // bm, n // bk)
    return pl.pallas_call(
        _decode_slab_body,
        grid=grid,
        in_specs=[
            pl.BlockSpec((bm, bk), lambda i, j: (i, j)),
            pl.BlockSpec((bm, 1), lambda i, j: (i, 0)),
            pl.BlockSpec((d, bk), lambda i, j: (0, j)),
            pl.BlockSpec((1, d), lambda i, j: (0, 0)),
            pl.BlockSpec(memory_space=pltpu.MemorySpace.HBM),
        ],
        out_specs=pl.BlockSpec((bm, d), lambda i, j: (i + roff, 0)),
        out_shape=jax.ShapeDtypeStruct((mfull, d), jnp.float32),
        input_output_aliases={4: 0},
    )(z, t, w_dec, b_dec, out_buf)


def kernel(x, W_enc, b_enc, W_dec, b_dec):
    m, act = x.shape
    dict_size = W_enc.shape[0]
    b_enc2 = b_enc.reshape(1, dict_size)
    b_dec2 = b_dec.reshape(1, act)

    bm_e = min(512, m)
    bn_e = min(1024, dict_size)
    bm_d = min(1024, m)
    bk_d = min(512, dict_size)

    if m % 8192 == 0 and dict_size % 2048 == 0:
        nslab = 4
        sm = m // nslab
        bme, bmd = min(bm_e, sm), min(bm_d, sm)
        outs = []
        for s in range(nslab):
            zs = _encode(x, W_enc, b_enc2, bme, bn_e,
                         sm=sm, roff=s * (sm // bme))
            ts = _sc_select(zs, K).reshape(sm, 1)
            outs.append(_decode(zs, ts, W_dec, b_dec2, bmd, bk_d))
        return jnp.concatenate(outs, axis=0)

    z = _encode(x, W_enc, b_enc2, bm_e, bn_e)
    t = _select(z, min(128, m), K)
    return _decode(z, t, W_dec, b_dec2, bm_d, bk_d)
